# Initial kernel scaffold; baseline (speedup 1.0000x reference)
import jax
import jax.numpy as jnp
from jax.experimental import pallas as pl


def kernel(unknown, known, unknow_feats, known_feats, W1, b1, W2, b2):
    raise NotImplementedError("write your pallas kernel here")


# speedup vs baseline: 30.1482x; 30.1482x over previous
import functools

import jax
import jax.numpy as jnp
from jax.experimental import pallas as pl


def _fp_block_kernel(u_ref, kt_ref, uf_ref, kf_ref,
                     w1a_ref, w1b_ref, b1_ref, w2_ref, b2_ref, o_ref,
                     *, nb: int, m: int):
    u = u_ref[0]
    dx = u[:, 0:1] - kt_ref[0, 0:1, :]
    dy = u[:, 1:2] - kt_ref[0, 1:2, :]
    dz = u[:, 2:3] - kt_ref[0, 2:3, :]
    d = dx * dx + dy * dy + dz * dz

    iota = jax.lax.broadcasted_iota(jnp.int32, (nb, m), 1)
    inf = jnp.float32(jnp.inf)

    def min3(dcur):
        mv = jnp.min(dcur, axis=1, keepdims=True)
        am = jnp.min(jnp.where(dcur == mv, iota, m), axis=1, keepdims=True)
        return mv, am

    m1, a1 = min3(d)
    d = jnp.where(iota == a1, inf, d)
    m2, a2 = min3(d)
    d = jnp.where(iota == a2, inf, d)
    m3, a3 = min3(d)

    r1 = 1.0 / (m1 + 1e-10)
    r2 = 1.0 / (m2 + 1e-10)
    r3 = 1.0 / (m3 + 1e-10)
    norm = r1 + r2 + r3
    w1 = r1 / norm
    w2 = r2 / norm
    w3 = r3 / norm

    zero = jnp.float32(0.0)
    wmat = (jnp.where(iota == a1, w1, zero)
            + jnp.where(iota == a2, w2, zero)
            + jnp.where(iota == a3, w3, zero))

    interp = jnp.dot(wmat, kf_ref[0], preferred_element_type=jnp.float32)
    h = jnp.dot(interp, w1a_ref[...], preferred_element_type=jnp.float32)
    h = h + jnp.dot(uf_ref[0], w1b_ref[...], preferred_element_type=jnp.float32)
    h = jnp.maximum(h + b1_ref[...], zero)
    o = jnp.dot(h, w2_ref[...], preferred_element_type=jnp.float32)
    o_ref[0] = jnp.maximum(o + b2_ref[...], zero)


def kernel(unknown, known, unknow_feats, known_feats, W1, b1, W2, b2):
    B, n, _ = unknown.shape
    m = known.shape[1]
    C1 = unknow_feats.shape[2]
    C2 = known_feats.shape[2]
    nb = 256

    known_t = jnp.transpose(known, (0, 2, 1))
    w1a = jnp.transpose(W1[:, :C2])
    w1b = jnp.transpose(W1[:, C2:])
    w2t = jnp.transpose(W2)
    b1r = b1.reshape(1, -1)
    b2r = b2.reshape(1, -1)

    grid = (B, n // nb)
    out = pl.pallas_call(
        functools.partial(_fp_block_kernel, nb=nb, m=m),
        grid=grid,
        in_specs=[
            pl.BlockSpec((1, nb, 3), lambda b, i: (b, i, 0)),
            pl.BlockSpec((1, 3, m), lambda b, i: (b, 0, 0)),
            pl.BlockSpec((1, nb, C1), lambda b, i: (b, i, 0)),
            pl.BlockSpec((1, m, C2), lambda b, i: (b, 0, 0)),
            pl.BlockSpec((C2, 256), lambda b, i: (0, 0)),
            pl.BlockSpec((C1, 256), lambda b, i: (0, 0)),
            pl.BlockSpec((1, 256), lambda b, i: (0, 0)),
            pl.BlockSpec((256, 128), lambda b, i: (0, 0)),
            pl.BlockSpec((1, 128), lambda b, i: (0, 0)),
        ],
        out_specs=pl.BlockSpec((1, nb, 128), lambda b, i: (b, i, 0)),
        out_shape=jax.ShapeDtypeStruct((B, n, 128), jnp.float32),
    )(unknown, known_t, unknow_feats, known_feats, w1a, w1b, b1r, w2t, b2r)
    return out


# speedup vs baseline: 31.1865x; 1.0344x over previous
import functools

import jax
import jax.numpy as jnp
from jax.experimental import pallas as pl


def _fp_block_kernel(u_ref, kt_ref, uf_ref, kf_ref,
                     w1a_ref, w1b_ref, b1_ref, w2_ref, b2_ref, o_ref,
                     *, nb: int, m: int):
    u = u_ref[0]
    kt = kt_ref[0]
    cross = jnp.dot(u, kt, preferred_element_type=jnp.float32,
                    precision=jax.lax.Precision.HIGHEST)
    unorm = jnp.sum(u * u, axis=1, keepdims=True)
    knorm = jnp.sum(kt * kt, axis=0, keepdims=True)
    zero = jnp.float32(0.0)
    d0 = jnp.maximum(unorm + knorm - 2.0 * cross, zero)

    inf = jnp.float32(jnp.inf)
    m1 = jnp.min(d0, axis=1, keepdims=True)
    dm = jnp.where(d0 == m1, inf, d0)
    m2 = jnp.min(dm, axis=1, keepdims=True)
    dm = jnp.where(dm == m2, inf, dm)
    m3 = jnp.min(dm, axis=1, keepdims=True)

    rw = 1.0 / (d0 + 1e-10)
    wmat = jnp.where(d0 <= m3, rw, zero)
    norm = jnp.sum(wmat, axis=1, keepdims=True)

    interp = jnp.dot(wmat, kf_ref[0], preferred_element_type=jnp.float32)
    interp = interp * (1.0 / norm)
    h = jnp.dot(interp, w1a_ref[...], preferred_element_type=jnp.float32)
    h = h + jnp.dot(uf_ref[0], w1b_ref[...], preferred_element_type=jnp.float32)
    h = jnp.maximum(h + b1_ref[...], zero)
    o = jnp.dot(h, w2_ref[...], preferred_element_type=jnp.float32)
    o_ref[0] = jnp.maximum(o + b2_ref[...], zero)


def kernel(unknown, known, unknow_feats, known_feats, W1, b1, W2, b2):
    B, n, _ = unknown.shape
    m = known.shape[1]
    C1 = unknow_feats.shape[2]
    C2 = known_feats.shape[2]
    nb = 256

    known_t = jnp.transpose(known, (0, 2, 1))
    w1a = jnp.transpose(W1[:, :C2])
    w1b = jnp.transpose(W1[:, C2:])
    w2t = jnp.transpose(W2)
    b1r = b1.reshape(1, -1)
    b2r = b2.reshape(1, -1)

    grid = (B, n // nb)
    out = pl.pallas_call(
        functools.partial(_fp_block_kernel, nb=nb, m=m),
        grid=grid,
        in_specs=[
            pl.BlockSpec((1, nb, 3), lambda b, i: (b, i, 0)),
            pl.BlockSpec((1, 3, m), lambda b, i: (b, 0, 0)),
            pl.BlockSpec((1, nb, C1), lambda b, i: (b, i, 0)),
            pl.BlockSpec((1, m, C2), lambda b, i: (b, 0, 0)),
            pl.BlockSpec((C2, 256), lambda b, i: (0, 0)),
            pl.BlockSpec((C1, 256), lambda b, i: (0, 0)),
            pl.BlockSpec((1, 256), lambda b, i: (0, 0)),
            pl.BlockSpec((256, 128), lambda b, i: (0, 0)),
            pl.BlockSpec((1, 128), lambda b, i: (0, 0)),
        ],
        out_specs=pl.BlockSpec((1, nb, 128), lambda b, i: (b, i, 0)),
        out_shape=jax.ShapeDtypeStruct((B, n, 128), jnp.float32),
    )(unknown, known_t, unknow_feats, known_feats, w1a, w1b, b1r, w2t, b2r)
    return out


# speedup vs baseline: 43.4290x; 1.3926x over previous
import functools

import jax
import jax.numpy as jnp
from jax.experimental import pallas as pl


def _fp_block_kernel(u_ref, kt_ref, uf_ref, kf_ref,
                     w1a_ref, w1b_ref, b1_ref, w2_ref, b2_ref, o_ref,
                     *, nb: int, m: int):
    u = u_ref[0]
    dx = u[:, 0:1] - kt_ref[0, 0:1, :]
    dy = u[:, 1:2] - kt_ref[0, 1:2, :]
    dz = u[:, 2:3] - kt_ref[0, 2:3, :]
    d0 = dx * dx + dy * dy + dz * dz
    zero = jnp.float32(0.0)

    inf = jnp.float32(jnp.inf)
    m1 = jnp.min(d0, axis=1, keepdims=True)
    dm = jnp.where(d0 == m1, inf, d0)
    m2 = jnp.min(dm, axis=1, keepdims=True)
    dm = jnp.where(dm == m2, inf, dm)
    m3 = jnp.min(dm, axis=1, keepdims=True)

    rw = 1.0 / (d0 + 1e-10)
    wmat = jnp.where(d0 <= m3, rw, zero)
    norm = jnp.sum(wmat, axis=1, keepdims=True)

    interp = jnp.dot(wmat, kf_ref[0], preferred_element_type=jnp.float32)
    interp = interp * (1.0 / norm)
    h = jnp.dot(interp, w1a_ref[...], preferred_element_type=jnp.float32)
    h = h + jnp.dot(uf_ref[0], w1b_ref[...], preferred_element_type=jnp.float32)
    h = jnp.maximum(h + b1_ref[...], zero)
    o = jnp.dot(h, w2_ref[...], preferred_element_type=jnp.float32)
    o_ref[0] = jnp.maximum(o + b2_ref[...], zero)


def kernel(unknown, known, unknow_feats, known_feats, W1, b1, W2, b2):
    B, n, _ = unknown.shape
    m = known.shape[1]
    C1 = unknow_feats.shape[2]
    C2 = known_feats.shape[2]
    nb = 256

    known_t = jnp.transpose(known, (0, 2, 1))
    w1a = jnp.transpose(W1[:, :C2])
    w1b = jnp.transpose(W1[:, C2:])
    w2t = jnp.transpose(W2)
    b1r = b1.reshape(1, -1)
    b2r = b2.reshape(1, -1)

    grid = (B, n // nb)
    out = pl.pallas_call(
        functools.partial(_fp_block_kernel, nb=nb, m=m),
        grid=grid,
        in_specs=[
            pl.BlockSpec((1, nb, 3), lambda b, i: (b, i, 0)),
            pl.BlockSpec((1, 3, m), lambda b, i: (b, 0, 0)),
            pl.BlockSpec((1, nb, C1), lambda b, i: (b, i, 0)),
            pl.BlockSpec((1, m, C2), lambda b, i: (b, 0, 0)),
            pl.BlockSpec((C2, 256), lambda b, i: (0, 0)),
            pl.BlockSpec((C1, 256), lambda b, i: (0, 0)),
            pl.BlockSpec((1, 256), lambda b, i: (0, 0)),
            pl.BlockSpec((256, 128), lambda b, i: (0, 0)),
            pl.BlockSpec((1, 128), lambda b, i: (0, 0)),
        ],
        out_specs=pl.BlockSpec((1, nb, 128), lambda b, i: (b, i, 0)),
        out_shape=jax.ShapeDtypeStruct((B, n, 128), jnp.float32),
    )(unknown, known_t, unknow_feats, known_feats, w1a, w1b, b1r, w2t, b2r)
    return out


# speedup vs baseline: 49.5875x; 1.1418x over previous
import functools

import jax
import jax.numpy as jnp
from jax.experimental import pallas as pl


def _fp_block_kernel(u_ref, kt_ref, uf_ref, kf_ref,
                     w1a_ref, w1b_ref, b1_ref, w2_ref, b2_ref, o_ref,
                     *, nb: int, m: int):
    u = u_ref[0]
    dx = u[:, 0:1] - kt_ref[0, 0:1, :]
    dy = u[:, 1:2] - kt_ref[0, 1:2, :]
    dz = u[:, 2:3] - kt_ref[0, 2:3, :]
    d0 = dx * dx + dy * dy + dz * dz
    zero = jnp.float32(0.0)

    inf = jnp.float32(jnp.inf)
    m1 = jnp.min(d0, axis=1, keepdims=True)
    dm = jnp.where(d0 == m1, inf, d0)
    m2 = jnp.min(dm, axis=1, keepdims=True)
    dm = jnp.where(dm == m2, inf, dm)
    m3 = jnp.min(dm, axis=1, keepdims=True)

    rw = 1.0 / (d0 + 1e-10)
    wmat = jnp.where(d0 <= m3, rw, zero)
    norm = jnp.sum(wmat, axis=1, keepdims=True)

    interp = jnp.dot(wmat, kf_ref[0], preferred_element_type=jnp.float32)
    interp = interp * (1.0 / norm)
    h = jnp.dot(interp, w1a_ref[...], preferred_element_type=jnp.float32)
    h = h + jnp.dot(uf_ref[0], w1b_ref[...], preferred_element_type=jnp.float32)
    h = jnp.maximum(h + b1_ref[...], zero)
    o = jnp.dot(h, w2_ref[...], preferred_element_type=jnp.float32)
    o_ref[0] = jnp.maximum(o + b2_ref[...], zero)


def kernel(unknown, known, unknow_feats, known_feats, W1, b1, W2, b2):
    B, n, _ = unknown.shape
    m = known.shape[1]
    C1 = unknow_feats.shape[2]
    C2 = known_feats.shape[2]
    nb = 512

    known_t = jnp.transpose(known, (0, 2, 1))
    w1a = jnp.transpose(W1[:, :C2])
    w1b = jnp.transpose(W1[:, C2:])
    w2t = jnp.transpose(W2)
    b1r = b1.reshape(1, -1)
    b2r = b2.reshape(1, -1)

    grid = (B, n // nb)
    out = pl.pallas_call(
        functools.partial(_fp_block_kernel, nb=nb, m=m),
        grid=grid,
        in_specs=[
            pl.BlockSpec((1, nb, 3), lambda b, i: (b, i, 0)),
            pl.BlockSpec((1, 3, m), lambda b, i: (b, 0, 0)),
            pl.BlockSpec((1, nb, C1), lambda b, i: (b, i, 0)),
            pl.BlockSpec((1, m, C2), lambda b, i: (b, 0, 0)),
            pl.BlockSpec((C2, 256), lambda b, i: (0, 0)),
            pl.BlockSpec((C1, 256), lambda b, i: (0, 0)),
            pl.BlockSpec((1, 256), lambda b, i: (0, 0)),
            pl.BlockSpec((256, 128), lambda b, i: (0, 0)),
            pl.BlockSpec((1, 128), lambda b, i: (0, 0)),
        ],
        out_specs=pl.BlockSpec((1, nb, 128), lambda b, i: (b, i, 0)),
        out_shape=jax.ShapeDtypeStruct((B, n, 128), jnp.float32),
    )(unknown, known_t, unknow_feats, known_feats, w1a, w1b, b1r, w2t, b2r)
    return out


# speedup vs baseline: 53.5119x; 1.0791x over previous
import functools

import jax
import jax.numpy as jnp
from jax.experimental import pallas as pl


def _fp_block_kernel(u_ref, kt_ref, uf_ref, kf_ref,
                     w1a_ref, w1b_ref, b1_ref, w2_ref, b2_ref, o_ref,
                     *, nb: int, m: int):
    u = u_ref[0]
    dx = u[:, 0:1] - kt_ref[0, 0:1, :]
    dy = u[:, 1:2] - kt_ref[0, 1:2, :]
    dz = u[:, 2:3] - kt_ref[0, 2:3, :]
    d0 = dx * dx + dy * dy + dz * dz
    zero = jnp.float32(0.0)

    inf = jnp.float32(jnp.inf)
    m1 = jnp.min(d0, axis=1, keepdims=True)
    dm = jnp.where(d0 == m1, inf, d0)
    m2 = jnp.min(dm, axis=1, keepdims=True)
    dm = jnp.where(dm == m2, inf, dm)
    m3 = jnp.min(dm, axis=1, keepdims=True)

    rw = 1.0 / (d0 + 1e-10)
    wmat = jnp.where(d0 <= m3, rw, zero)
    norm = jnp.sum(wmat, axis=1, keepdims=True)

    interp = jnp.dot(wmat, kf_ref[0], preferred_element_type=jnp.float32)
    interp = interp * (1.0 / norm)
    h = jnp.dot(interp, w1a_ref[...], preferred_element_type=jnp.float32)
    h = h + jnp.dot(uf_ref[0], w1b_ref[...], preferred_element_type=jnp.float32)
    h = jnp.maximum(h + b1_ref[...], zero)
    o = jnp.dot(h, w2_ref[...], preferred_element_type=jnp.float32)
    o_ref[0] = jnp.maximum(o + b2_ref[...], zero)


def kernel(unknown, known, unknow_feats, known_feats, W1, b1, W2, b2):
    B, n, _ = unknown.shape
    m = known.shape[1]
    C1 = unknow_feats.shape[2]
    C2 = known_feats.shape[2]
    nb = 1024

    known_t = jnp.transpose(known, (0, 2, 1))
    w1a = jnp.transpose(W1[:, :C2])
    w1b = jnp.transpose(W1[:, C2:])
    w2t = jnp.transpose(W2)
    b1r = b1.reshape(1, -1)
    b2r = b2.reshape(1, -1)

    grid = (B, n // nb)
    out = pl.pallas_call(
        functools.partial(_fp_block_kernel, nb=nb, m=m),
        grid=grid,
        in_specs=[
            pl.BlockSpec((1, nb, 3), lambda b, i: (b, i, 0)),
            pl.BlockSpec((1, 3, m), lambda b, i: (b, 0, 0)),
            pl.BlockSpec((1, nb, C1), lambda b, i: (b, i, 0)),
            pl.BlockSpec((1, m, C2), lambda b, i: (b, 0, 0)),
            pl.BlockSpec((C2, 256), lambda b, i: (0, 0)),
            pl.BlockSpec((C1, 256), lambda b, i: (0, 0)),
            pl.BlockSpec((1, 256), lambda b, i: (0, 0)),
            pl.BlockSpec((256, 128), lambda b, i: (0, 0)),
            pl.BlockSpec((1, 128), lambda b, i: (0, 0)),
        ],
        out_specs=pl.BlockSpec((1, nb, 128), lambda b, i: (b, i, 0)),
        out_shape=jax.ShapeDtypeStruct((B, n, 128), jnp.float32),
    )(unknown, known_t, unknow_feats, known_feats, w1a, w1b, b1r, w2t, b2r)
    return out


# speedup vs baseline: 56.8602x; 1.0626x over previous
import functools

import jax
import jax.numpy as jnp
from jax.experimental import pallas as pl


def _fp_block_kernel(u_ref, kt_ref, uf_ref, kf_ref,
                     w1a_ref, w1b_ref, b1_ref, w2_ref, b2_ref, o_ref,
                     *, nb: int, m: int):
    u = u_ref[0]
    dx = u[:, 0:1] - kt_ref[0, 0:1, :]
    dy = u[:, 1:2] - kt_ref[0, 1:2, :]
    dz = u[:, 2:3] - kt_ref[0, 2:3, :]
    d0 = dx * dx + dy * dy + dz * dz
    zero = jnp.float32(0.0)

    inf = jnp.float32(jnp.inf)
    m1 = jnp.min(d0, axis=1, keepdims=True)
    dm = jnp.where(d0 == m1, inf, d0)
    m2 = jnp.min(dm, axis=1, keepdims=True)
    dm = jnp.where(dm == m2, inf, dm)
    m3 = jnp.min(dm, axis=1, keepdims=True)

    rw = 1.0 / (d0 + 1e-10)
    wmat = jnp.where(d0 <= m3, rw, zero)
    norm = jnp.sum(wmat, axis=1, keepdims=True)

    interp = jnp.dot(wmat, kf_ref[0], preferred_element_type=jnp.float32)
    interp = interp * (1.0 / norm)
    h = jnp.dot(interp, w1a_ref[...], preferred_element_type=jnp.float32)
    h = h + jnp.dot(uf_ref[0], w1b_ref[...], preferred_element_type=jnp.float32)
    h = jnp.maximum(h + b1_ref[...], zero)
    o = jnp.dot(h, w2_ref[...], preferred_element_type=jnp.float32)
    o_ref[0] = jnp.maximum(o + b2_ref[...], zero)


def kernel(unknown, known, unknow_feats, known_feats, W1, b1, W2, b2):
    B, n, _ = unknown.shape
    m = known.shape[1]
    C1 = unknow_feats.shape[2]
    C2 = known_feats.shape[2]
    nb = 2048

    known_t = jnp.transpose(known, (0, 2, 1))
    w1a = jnp.transpose(W1[:, :C2])
    w1b = jnp.transpose(W1[:, C2:])
    w2t = jnp.transpose(W2)
    b1r = b1.reshape(1, -1)
    b2r = b2.reshape(1, -1)

    grid = (B, n // nb)
    out = pl.pallas_call(
        functools.partial(_fp_block_kernel, nb=nb, m=m),
        grid=grid,
        in_specs=[
            pl.BlockSpec((1, nb, 3), lambda b, i: (b, i, 0)),
            pl.BlockSpec((1, 3, m), lambda b, i: (b, 0, 0)),
            pl.BlockSpec((1, nb, C1), lambda b, i: (b, i, 0)),
            pl.BlockSpec((1, m, C2), lambda b, i: (b, 0, 0)),
            pl.BlockSpec((C2, 256), lambda b, i: (0, 0)),
            pl.BlockSpec((C1, 256), lambda b, i: (0, 0)),
            pl.BlockSpec((1, 256), lambda b, i: (0, 0)),
            pl.BlockSpec((256, 128), lambda b, i: (0, 0)),
            pl.BlockSpec((1, 128), lambda b, i: (0, 0)),
        ],
        out_specs=pl.BlockSpec((1, nb, 128), lambda b, i: (b, i, 0)),
        out_shape=jax.ShapeDtypeStruct((B, n, 128), jnp.float32),
    )(unknown, known_t, unknow_feats, known_feats, w1a, w1b, b1r, w2t, b2r)
    return out


# speedup vs baseline: 62.6946x; 1.1026x over previous
import functools

import jax
import jax.numpy as jnp
from jax.experimental import pallas as pl


def _fp_block_kernel(u_ref, kt_ref, uf_ref, kf_ref,
                     w1a_ref, w1b_ref, b1_ref, w2_ref, b2_ref, o_ref,
                     *, nb: int, m: int):
    u = u_ref[0]
    dx = u[:, 0:1] - kt_ref[0, 0:1, :]
    dy = u[:, 1:2] - kt_ref[0, 1:2, :]
    dz = u[:, 2:3] - kt_ref[0, 2:3, :]
    d0 = dx * dx + dy * dy + dz * dz
    zero = jnp.float32(0.0)

    inf = jnp.float32(jnp.inf)
    m1 = jnp.min(d0, axis=1, keepdims=True)
    dm = jnp.where(d0 == m1, inf, d0)
    m2 = jnp.min(dm, axis=1, keepdims=True)
    dm = jnp.where(dm == m2, inf, dm)
    m3 = jnp.min(dm, axis=1, keepdims=True)

    rw = 1.0 / (d0 + 1e-10)
    wmat = jnp.where(d0 <= m3, rw, zero)
    norm = jnp.sum(wmat, axis=1, keepdims=True)

    interp = jnp.dot(wmat, kf_ref[0], preferred_element_type=jnp.float32)
    interp = interp * (1.0 / norm)
    h = jnp.dot(interp, w1a_ref[...], preferred_element_type=jnp.float32)
    h = h + jnp.dot(uf_ref[0], w1b_ref[...], preferred_element_type=jnp.float32)
    h = jnp.maximum(h + b1_ref[...], zero)
    o = jnp.dot(h, w2_ref[...], preferred_element_type=jnp.float32)
    o_ref[0] = jnp.maximum(o + b2_ref[...], zero)


def kernel(unknown, known, unknow_feats, known_feats, W1, b1, W2, b2):
    B, n, _ = unknown.shape
    m = known.shape[1]
    C1 = unknow_feats.shape[2]
    C2 = known_feats.shape[2]
    nb = 4096

    known_t = jnp.transpose(known, (0, 2, 1))
    w1a = jnp.transpose(W1[:, :C2])
    w1b = jnp.transpose(W1[:, C2:])
    w2t = jnp.transpose(W2)
    b1r = b1.reshape(1, -1)
    b2r = b2.reshape(1, -1)

    grid = (B, n // nb)
    out = pl.pallas_call(
        functools.partial(_fp_block_kernel, nb=nb, m=m),
        grid=grid,
        in_specs=[
            pl.BlockSpec((1, nb, 3), lambda b, i: (b, i, 0)),
            pl.BlockSpec((1, 3, m), lambda b, i: (b, 0, 0)),
            pl.BlockSpec((1, nb, C1), lambda b, i: (b, i, 0)),
            pl.BlockSpec((1, m, C2), lambda b, i: (b, 0, 0)),
            pl.BlockSpec((C2, 256), lambda b, i: (0, 0)),
            pl.BlockSpec((C1, 256), lambda b, i: (0, 0)),
            pl.BlockSpec((1, 256), lambda b, i: (0, 0)),
            pl.BlockSpec((256, 128), lambda b, i: (0, 0)),
            pl.BlockSpec((1, 128), lambda b, i: (0, 0)),
        ],
        out_specs=pl.BlockSpec((1, nb, 128), lambda b, i: (b, i, 0)),
        out_shape=jax.ShapeDtypeStruct((B, n, 128), jnp.float32),
    )(unknown, known_t, unknow_feats, known_feats, w1a, w1b, b1r, w2t, b2r)
    return out


# speedup vs baseline: 66.3028x; 1.0576x over previous
import functools

import jax
import jax.numpy as jnp
from jax.experimental import pallas as pl


def _fp_block_kernel(u_ref, kt_ref, uf_ref, kf_ref,
                     w1a_ref, w1b_ref, b1_ref, w2_ref, b2_ref, o_ref,
                     *, nb: int, m: int):
    u = u_ref[0]
    kt = kt_ref[0]
    bf16 = jnp.bfloat16
    f32 = jnp.float32

    uh = u.astype(bf16)
    ur = u - uh.astype(f32)
    um = ur.astype(bf16)
    ul = (ur - um.astype(f32)).astype(bf16)
    kh = kt.astype(bf16)
    kr = kt - kh.astype(f32)
    km = kr.astype(bf16)
    kl = (kr - km.astype(f32)).astype(bf16)
    lhs = jnp.concatenate([uh, uh, um, uh, ul, um], axis=1)
    rhs = jnp.concatenate([kh, km, kh, kl, kh, km], axis=0)
    cross = jnp.dot(lhs, rhs, preferred_element_type=f32)

    unorm = jnp.sum(u * u, axis=1, keepdims=True)
    knorm = jnp.sum(kt * kt, axis=0, keepdims=True)
    zero = jnp.float32(0.0)
    d0 = jnp.maximum((unorm + knorm) - (cross + cross), zero)

    inf = jnp.float32(jnp.inf)
    m1 = jnp.min(d0, axis=1, keepdims=True)
    dm = jnp.where(d0 == m1, inf, d0)
    m2 = jnp.min(dm, axis=1, keepdims=True)
    dm = jnp.where(dm == m2, inf, dm)
    m3 = jnp.min(dm, axis=1, keepdims=True)

    r1 = 1.0 / (m1 + 1e-10)
    r2 = 1.0 / (m2 + 1e-10)
    r3 = 1.0 / (m3 + 1e-10)
    rnorm = 1.0 / (r1 + r2 + r3)
    sel = jnp.where(d0 <= m3, d0, inf)
    wmat = 1.0 / (sel + 1e-10)

    interp = jnp.dot(wmat, kf_ref[0], preferred_element_type=jnp.float32)
    interp = interp * rnorm
    h = jnp.dot(interp, w1a_ref[...], preferred_element_type=jnp.float32)
    h = h + jnp.dot(uf_ref[0], w1b_ref[...], preferred_element_type=jnp.float32)
    h = jnp.maximum(h + b1_ref[...], zero)
    o = jnp.dot(h, w2_ref[...], preferred_element_type=jnp.float32)
    o_ref[0] = jnp.maximum(o + b2_ref[...], zero)


def kernel(unknown, known, unknow_feats, known_feats, W1, b1, W2, b2):
    B, n, _ = unknown.shape
    m = known.shape[1]
    C1 = unknow_feats.shape[2]
    C2 = known_feats.shape[2]
    nb = min(4096, n)

    known_t = jnp.transpose(known, (0, 2, 1))
    w1a = jnp.transpose(W1[:, :C2])
    w1b = jnp.transpose(W1[:, C2:])
    w2t = jnp.transpose(W2)
    b1r = b1.reshape(1, -1)
    b2r = b2.reshape(1, -1)

    grid = (B, n // nb)
    out = pl.pallas_call(
        functools.partial(_fp_block_kernel, nb=nb, m=m),
        grid=grid,
        in_specs=[
            pl.BlockSpec((1, nb, 3), lambda b, i: (b, i, 0)),
            pl.BlockSpec((1, 3, m), lambda b, i: (b, 0, 0)),
            pl.BlockSpec((1, nb, C1), lambda b, i: (b, i, 0)),
            pl.BlockSpec((1, m, C2), lambda b, i: (b, 0, 0)),
            pl.BlockSpec((C2, 256), lambda b, i: (0, 0)),
            pl.BlockSpec((C1, 256), lambda b, i: (0, 0)),
            pl.BlockSpec((1, 256), lambda b, i: (0, 0)),
            pl.BlockSpec((256, 128), lambda b, i: (0, 0)),
            pl.BlockSpec((1, 128), lambda b, i: (0, 0)),
        ],
        out_specs=pl.BlockSpec((1, nb, 128), lambda b, i: (b, i, 0)),
        out_shape=jax.ShapeDtypeStruct((B, n, 128), jnp.float32),
    )(unknown, known_t, unknow_feats, known_feats, w1a, w1b, b1r, w2t, b2r)
    return out
